# Initial kernel scaffold; baseline (speedup 1.0000x reference)
import jax
import jax.numpy as jnp
from jax.experimental import pallas as pl


def kernel(x, ffn_norm_w, gate_w, up_w, down_w, shared_up_w, shared_down_w):
    raise NotImplementedError("write your pallas kernel here")


# speedup vs baseline: 1.1266x; 1.1266x over previous
import functools

import jax
import jax.numpy as jnp
from jax.experimental import pallas as pl
from jax.experimental.pallas import tpu as pltpu

_H = 1024
_E = 8
_FF = 1024
_SFF = 2048
_T = 2048
_EPS = 1e-6


def _router_body(x_ref, nw_ref, gw_ref, h_ref, cmb_ref):
    x = x_ref[...]
    var = jnp.mean(x * x, axis=-1, keepdims=True)
    h = x * jax.lax.rsqrt(var + _EPS) * nw_ref[...]
    h_ref[...] = h
    logits = jax.lax.dot_general(h, gw_ref[...], (((1,), (1,)), ((), ())),
                                 preferred_element_type=jnp.float32)
    m = jnp.max(logits, axis=-1, keepdims=True)
    ex = jnp.exp(logits - m)
    p = ex / jnp.sum(ex, axis=-1, keepdims=True)
    lanes = jax.lax.broadcasted_iota(jnp.int32, p.shape, 1)
    v1 = jnp.max(p, axis=-1, keepdims=True)
    i1 = jnp.min(jnp.where(p == v1, lanes, _E), axis=-1, keepdims=True)
    p2 = jnp.where(lanes == i1, -jnp.inf, p)
    v2 = jnp.max(p2, axis=-1, keepdims=True)
    i2 = jnp.min(jnp.where(p2 == v2, lanes, _E), axis=-1, keepdims=True)
    s = v1 + v2
    cmb_ref[...] = (jnp.where(lanes == i1, v1 / s, 0.0)
                    + jnp.where(lanes == i2, v2 / s, 0.0))


def _moe_body(h_ref, up_ref, dn_ref, cmb_ref, o_ref):
    e = pl.program_id(1)
    u = jax.lax.dot_general(h_ref[...], up_ref[0], (((1,), (1,)), ((), ())),
                            preferred_element_type=jnp.float32)
    g = u[:, :_FF]
    v = u[:, _FF:]
    act = (g / (1.0 + jnp.exp(-g))) * v
    y = jax.lax.dot_general(act, dn_ref[0], (((1,), (1,)), ((), ())),
                            preferred_element_type=jnp.float32)
    lanes = jax.lax.broadcasted_iota(jnp.int32, (1, _E), 1)
    w = jnp.sum(jnp.where(lanes == e, cmb_ref[...], 0.0), axis=1, keepdims=True)
    contrib = y * w

    @pl.when(e == 0)
    def _init():
        o_ref[...] = contrib

    @pl.when(e > 0)
    def _acc():
        o_ref[...] += contrib


def _shared_body(x_ref, h_ref, moe_ref, su_ref, sd_ref, o_ref):
    u = jax.lax.dot_general(h_ref[...], su_ref[...], (((1,), (1,)), ((), ())),
                            preferred_element_type=jnp.float32)
    g = u[:, :_SFF]
    v = u[:, _SFF:]
    act = (g / (1.0 + jnp.exp(-g))) * v
    y = jax.lax.dot_general(act, sd_ref[...], (((1,), (1,)), ((), ())),
                            preferred_element_type=jnp.float32)
    o_ref[...] = x_ref[...] + moe_ref[...] + y


@jax.jit
def kernel(x, ffn_norm_w, gate_w, up_w, down_w, shared_up_w, shared_down_w):
    nw = ffn_norm_w.reshape(1, _H)

    h, cmb = pl.pallas_call(
        _router_body,
        out_shape=[
            jax.ShapeDtypeStruct((_T, _H), jnp.float32),
            jax.ShapeDtypeStruct((_T, _E), jnp.float32),
        ],
    )(x, nw, gate_w)

    tb = 512
    moe = pl.pallas_call(
        _moe_body,
        grid=(_T // tb, _E),
        in_specs=[
            pl.BlockSpec((tb, _H), lambda t, e: (t, 0)),
            pl.BlockSpec((1, 2 * _FF, _H), lambda t, e: (e, 0, 0)),
            pl.BlockSpec((1, _H, _FF), lambda t, e: (e, 0, 0)),
            pl.BlockSpec((tb, _E), lambda t, e: (t, 0)),
        ],
        out_specs=pl.BlockSpec((tb, _H), lambda t, e: (t, 0)),
        out_shape=jax.ShapeDtypeStruct((_T, _H), jnp.float32),
    )(h, up_w, down_w, cmb)

    sb = 256
    out = pl.pallas_call(
        _shared_body,
        grid=(_T // sb,),
        in_specs=[
            pl.BlockSpec((sb, _H), lambda t: (t, 0)),
            pl.BlockSpec((sb, _H), lambda t: (t, 0)),
            pl.BlockSpec((sb, _H), lambda t: (t, 0)),
            pl.BlockSpec((2 * _SFF, _H), lambda t: (0, 0)),
            pl.BlockSpec((_H, _SFF), lambda t: (0, 0)),
        ],
        out_specs=pl.BlockSpec((sb, _H), lambda t: (t, 0)),
        out_shape=jax.ShapeDtypeStruct((_T, _H), jnp.float32),
    )(x, h, moe, shared_up_w, shared_down_w)
    return out
